# baseline (speedup 1.0000x reference)
import functools

import jax
import jax.numpy as jnp
from jax import lax
from jax.experimental import pallas as pl
from jax.experimental.pallas import tpu as pltpu
from jax.experimental.pallas import tpu_sc as plsc

F32 = jnp.float32
I32 = jnp.int32

N = 10000
NP = 10240
E_IN = 320000
ETOT = E_IN + N
NC, NS = 2, 16
NW = NC * NS
B = 128
K = -(-ETOT // (NW * B))
KA = 100
KB = 64
KMAX = max(KA, KB)
EPAD = NS * B * (KA + KB)
RW1 = 80
RW2 = 48
NEG = 0.2
BLK = 512
RPT = NP // NS


def _tc1_body(x_ref, w1_ref, as_ref, ad_ref, t_ref, aldt_ref):
    h = jnp.dot(x_ref[...], w1_ref[...], preferred_element_type=F32)
    als = jnp.dot(h, as_ref[...], preferred_element_type=F32)
    ald = jnp.dot(h, ad_ref[...], preferred_element_type=F32)
    t_ref[...] = jnp.concatenate([h, als, als], axis=1)
    aldt_ref[...] = jnp.concatenate([ald, ald], axis=1)


def _tc2_body(p_ref, b1_ref, w2_ref, as2_ref, ad2_ref, r8_ref, t2_ref, ald2_ref):
    a = p_ref[0] + p_ref[1]
    den64 = jnp.dot(a[:, 64:72], r8_ref[...], preferred_element_type=F32)
    x2 = a[:, 0:64] / (den64 + 1e-16) + b1_ref[...]
    x2 = jnp.where(x2 > 0, x2, jnp.exp(x2) - 1.0)
    h2 = jnp.dot(x2, w2_ref[...], preferred_element_type=F32)
    als2 = jnp.dot(h2, as2_ref[...], preferred_element_type=F32)
    ald2 = jnp.dot(h2, ad2_ref[...], preferred_element_type=F32)
    t2_ref[...] = jnp.concatenate([h2, als2], axis=1)
    ald2_ref[...] = ald2


def _tc3_body(p_ref, b2_ref, rb_ref, o_ref):
    a = p_ref[0] + p_ref[1]
    den40 = jnp.dot(a[:, 40:48], rb_ref[...], preferred_element_type=F32)
    o_ref[...] = a[:, 0:40] / (den40 + 1e-16) + b2_ref[...] + 1e-6


def _sc_body(layer, sdidx, table, aldt, out, idx_all, r0, r1, r2, r3,
             a0, a1, a2, a3, acc_sh, tbl_sh, g0, g1, g2, g3,
             s0, s1, s2, s3):
    rw = RW1 if layer == 1 else RW2
    stage = False
    rows = (r0, r1, r2, r3)
    alds = (a0, a1, a2, a3)
    gsem = (g0, g1, g2, g3)
    ssem = (s0, s1, s2, s3)
    c = lax.axis_index("c")
    s = lax.axis_index("s")
    w = c * NS + s
    rowbase = s * RPT

    pltpu.sync_copy(sdidx.at[w], idx_all)
    if stage:
        pltpu.sync_copy(table.at[pl.ds(rowbase, RPT)],
                        tbl_sh.at[pl.ds(rowbase, RPT)])
    src_tbl = tbl_sh if stage else table

    zero16 = jnp.zeros((16,), F32)

    def zrow(e, carry):
        for j in range(rw // 16):
            r0[e, pl.ds(16 * j, 16)] = zero16
        return carry

    lax.fori_loop(0, B, zrow, 0)
    for t in range(RPT // B):
        pltpu.sync_copy(r0, acc_sh.at[pl.ds(rowbase + t * B, B)])

    def start_gather(k, j):
        pltpu.async_copy(src_tbl.at[idx_all.at[k, 0]], rows[j], gsem[j])
        pltpu.async_copy(aldt.at[idx_all.at[k, 1]], alds[j], gsem[j])

    def wait_gather(j):
        pltpu.make_async_copy(src_tbl.at[pl.ds(0, B)], rows[j],
                              gsem[j]).wait()
        pltpu.make_async_copy(aldt.at[pl.ds(0, B)], alds[j], gsem[j]).wait()

    def start_scatter(k, j):
        pltpu.async_copy(rows[j], acc_sh.at[idx_all.at[k, 1]], ssem[j],
                         add=True)

    def wait_scatter(j):
        pltpu.make_async_copy(rows[j], acc_sh.at[pl.ds(0, B)], ssem[j]).wait()

    plsc.subcore_barrier()
    start_gather(0, 0)
    start_gather(1, 1)

    iota = lax.iota(I32, 16)
    base8 = lax.shift_right_logical(iota, 3)

    def compute(j):
        rv = rows[j]
        av = alds[j]
        if layer == 1:
            @plsc.parallel_loop(0, B, 1, unroll=4)
            def edge(e):
                ald16 = av[e, :]
                srcal = rv[e, pl.ds(64, 16)]
                am = srcal + ald16
                am = jnp.maximum(am, am * NEG)
                ex = jnp.exp(am)
                rv[e, pl.ds(64, 16)] = ex
                for jj in range(4):
                    mult = jnp.take(ex, base8 + 2 * jj)
                    hj = rv[e, pl.ds(16 * jj, 16)]
                    rv[e, pl.ds(16 * jj, 16)] = hj * mult
        else:
            c12 = jnp.full((16,), 12, I32)

            @plsc.parallel_loop(0, B, 1, unroll=4)
            def edge(e):
                ald16 = av[e, :]
                t16 = rv[e, pl.ds(32, 16)]
                am = t16 + ald16
                am = jnp.maximum(am, am * NEG)
                ex = jnp.exp(am)
                mult = jnp.take(ex, c12)
                h0 = rv[e, pl.ds(0, 16)]
                rv[e, pl.ds(0, 16)] = h0 * mult
                h1 = rv[e, pl.ds(16, 16)]
                rv[e, pl.ds(16, 16)] = h1 * mult
                rv[e, pl.ds(32, 16)] = jnp.where(iota < 8, t16 * mult, mult)

    kw = jnp.where(c == 0, KA, KB)

    def body_t(t, carry):
        for j in range(4):
            k = 4 * t + j
            nb = (j + 2) % 4

            @pl.when(k + 2 < kw)
            def _prefetch():
                @pl.when(k >= 2)
                def _drain():
                    wait_scatter(nb)

                start_gather(k + 2, nb)

            wait_gather(j)
            compute(j)
            start_scatter(k, j)
        return carry

    lax.fori_loop(0, kw // 4, body_t, 0)
    for j in range(4):
        wait_scatter(j)
    plsc.subcore_barrier()
    for t in range(RPT // B):
        pltpu.sync_copy(acc_sh.at[pl.ds(rowbase + t * B, B)], r0)
        pltpu.sync_copy(r0, out.at[c, pl.ds(rowbase + t * B, B)])


def _sc_edge(layer):
    rw = RW1 if layer == 1 else RW2
    mesh = plsc.VectorSubcoreMesh(core_axis_name="c", subcore_axis_name="s")
    return pl.kernel(
        functools.partial(_sc_body, layer),
        out_type=jax.ShapeDtypeStruct((NC, NP, rw), F32),
        mesh=mesh,
        scratch_types=[
            pltpu.VMEM((KMAX, 2, B), I32),
            pltpu.VMEM((B, rw), F32),
            pltpu.VMEM((B, rw), F32),
            pltpu.VMEM((B, rw), F32),
            pltpu.VMEM((B, rw), F32),
            pltpu.VMEM((B, 16), F32),
            pltpu.VMEM((B, 16), F32),
            pltpu.VMEM((B, 16), F32),
            pltpu.VMEM((B, 16), F32),
            pltpu.VMEM_SHARED((NP, rw), F32),
            pltpu.VMEM_SHARED((8, rw), F32),
        ] + [pltpu.SemaphoreType.DMA] * 8,
        compiler_params=pltpu.CompilerParams(
            needs_layout_passes=False, use_tc_tiling_on_sc=False),
    )


def kernel(x, edge_index, W1, att_src1, att_dst1, b1, W2, att_src2, att_dst2,
           b2):
    loop = jnp.arange(N, dtype=I32)
    padv = jnp.full((EPAD - ETOT,), N, I32)
    sidx = jnp.concatenate([edge_index[0].astype(I32), loop, padv])
    didx = jnp.concatenate([edge_index[1].astype(I32), loop, padv])

    def _split(v):
        ec0 = NS * KA * B
        v0 = v[:ec0].reshape(NS, KA, B)
        v0 = jnp.pad(v0, ((0, 0), (0, KMAX - KA), (0, 0)),
                     constant_values=N)
        v1 = v[ec0:].reshape(NS, KB, B)
        v1 = jnp.pad(v1, ((0, 0), (0, KMAX - KB), (0, 0)),
                     constant_values=N)
        return jnp.concatenate([v0, v1], axis=0)

    sdidx = jnp.stack([_split(sidx), _split(didx)], axis=2)
    xp = jnp.pad(x, ((0, NP - N), (0, 0)))

    eye8 = jnp.eye(8, dtype=F32)
    as64 = (att_src1[:, :, None] * eye8[:, None, :]).reshape(64, 8)
    ad64 = (att_dst1[:, :, None] * eye8[:, None, :]).reshape(64, 8)

    table1, aldt1 = pl.pallas_call(
        _tc1_body,
        grid=(NP // BLK,),
        in_specs=[
            pl.BlockSpec((BLK, 128), lambda i: (i, 0)),
            pl.BlockSpec((128, 64), lambda i: (0, 0)),
            pl.BlockSpec((64, 8), lambda i: (0, 0)),
            pl.BlockSpec((64, 8), lambda i: (0, 0)),
        ],
        out_specs=[
            pl.BlockSpec((BLK, RW1), lambda i: (i, 0)),
            pl.BlockSpec((BLK, 16), lambda i: (i, 0)),
        ],
        out_shape=[
            jax.ShapeDtypeStruct((NP, RW1), F32),
            jax.ShapeDtypeStruct((NP, 16), F32),
        ],
    )(xp, W1, as64, ad64)

    part1 = _sc_edge(1)(sdidx, table1, aldt1)

    r8 = jnp.repeat(jnp.eye(8, dtype=F32), 8, axis=1)
    as2t = jnp.tile(att_src2.T, (1, 8))
    ad2t = jnp.tile(att_dst2.T, (1, 16))

    table2, aldt2 = pl.pallas_call(
        _tc2_body,
        grid=(NP // BLK,),
        in_specs=[
            pl.BlockSpec((NC, BLK, RW1), lambda i: (0, i, 0)),
            pl.BlockSpec((1, 64), lambda i: (0, 0)),
            pl.BlockSpec((64, 40), lambda i: (0, 0)),
            pl.BlockSpec((40, 8), lambda i: (0, 0)),
            pl.BlockSpec((40, 16), lambda i: (0, 0)),
            pl.BlockSpec((8, 64), lambda i: (0, 0)),
        ],
        out_specs=[
            pl.BlockSpec((BLK, RW2), lambda i: (i, 0)),
            pl.BlockSpec((BLK, 16), lambda i: (i, 0)),
        ],
        out_shape=[
            jax.ShapeDtypeStruct((NP, RW2), F32),
            jax.ShapeDtypeStruct((NP, 16), F32),
        ],
    )(part1, b1.reshape(1, 64), W2, as2t, ad2t, r8)

    part2 = _sc_edge(2)(sdidx, table2, aldt2)

    rb = jnp.concatenate([jnp.ones((1, 40), F32), jnp.zeros((7, 40), F32)])
    out = pl.pallas_call(
        _tc3_body,
        grid=(25,),
        in_specs=[
            pl.BlockSpec((NC, 400, RW2), lambda i: (0, i, 0)),
            pl.BlockSpec((1, 40), lambda i: (0, 0)),
            pl.BlockSpec((8, 40), lambda i: (0, 0)),
        ],
        out_specs=pl.BlockSpec((400, 40), lambda i: (i, 0)),
        out_shape=jax.ShapeDtypeStruct((N, 40), F32),
    )(part2, b2.reshape(1, 40), rb)
    return out


# speedup vs baseline: 1.5849x; 1.5849x over previous
import functools

import jax
import jax.numpy as jnp
from jax import lax
from jax.experimental import pallas as pl
from jax.experimental.pallas import tpu as pltpu
from jax.experimental.pallas import tpu_sc as plsc

F32 = jnp.float32
I32 = jnp.int32

N = 10000
NP = 10240
E_IN = 320000
ETOT = E_IN + N
NC, NS = 2, 16
NW = NC * NS
B = 128
K = -(-ETOT // (NW * B))
KA = 99
KB = 2 * K - KA
KMAX = max(KA, KB)
EPAD = NW * B * K
RW1 = 80
RW2 = 48
NEG = 0.2
BLK = 512
RPT = NP // NS


def _tc1_body(x_ref, w1_ref, as_ref, ad_ref, t_ref, aldt_ref):
    h = jnp.dot(x_ref[...], w1_ref[...], preferred_element_type=F32)
    als = jnp.dot(h, as_ref[...], preferred_element_type=F32)
    ald = jnp.dot(h, ad_ref[...], preferred_element_type=F32)
    t_ref[...] = jnp.concatenate([h, als, als], axis=1)
    aldt_ref[...] = jnp.concatenate([ald, ald], axis=1)


def _tc2_body(p_ref, b1_ref, w2_ref, as2_ref, ad2_ref, r8_ref, t2_ref, ald2_ref):
    a = p_ref[0] + p_ref[1]
    den64 = jnp.dot(a[:, 64:72], r8_ref[...], preferred_element_type=F32)
    x2 = a[:, 0:64] / (den64 + 1e-16) + b1_ref[...]
    x2 = jnp.where(x2 > 0, x2, jnp.exp(x2) - 1.0)
    h2 = jnp.dot(x2, w2_ref[...], preferred_element_type=F32)
    als2 = jnp.dot(h2, as2_ref[...], preferred_element_type=F32)
    ald2 = jnp.dot(h2, ad2_ref[...], preferred_element_type=F32)
    t2_ref[...] = jnp.concatenate([h2, als2], axis=1)
    ald2_ref[...] = ald2


def _tc3_body(p_ref, b2_ref, rb_ref, o_ref):
    a = p_ref[0] + p_ref[1]
    den40 = jnp.dot(a[:, 40:48], rb_ref[...], preferred_element_type=F32)
    o_ref[...] = a[:, 0:40] / (den40 + 1e-16) + b2_ref[...] + 1e-6


def _sc_body(layer, sdidx, table, aldt, out, idx_all, r0, r1, r2, a0, a1, a2,
             acc_sh, g0, g1, g2, s0, s1, s2):
    rw = RW1 if layer == 1 else RW2
    rows = (r0, r1, r2)
    alds = (a0, a1, a2)
    gsem = (g0, g1, g2)
    ssem = (s0, s1, s2)
    c = lax.axis_index("c")
    s = lax.axis_index("s")
    w = c * NS + s
    rowbase = s * RPT

    pltpu.sync_copy(sdidx.at[w], idx_all)

    zero16 = jnp.zeros((16,), F32)

    def zrow(e, carry):
        for j in range(rw // 16):
            r0[e, pl.ds(16 * j, 16)] = zero16
        return carry

    lax.fori_loop(0, B, zrow, 0)
    for t in range(RPT // B):
        pltpu.sync_copy(r0, acc_sh.at[pl.ds(rowbase + t * B, B)])

    def start_gather(k, j):
        pltpu.async_copy(table.at[idx_all.at[k, 0]], rows[j], gsem[j])
        pltpu.async_copy(aldt.at[idx_all.at[k, 1]], alds[j], gsem[j])

    def wait_gather(j):
        pltpu.make_async_copy(table.at[pl.ds(0, B)], rows[j],
                              gsem[j]).wait()
        pltpu.make_async_copy(aldt.at[pl.ds(0, B)], alds[j], gsem[j]).wait()

    def start_scatter(k, j):
        pltpu.async_copy(rows[j], acc_sh.at[idx_all.at[k, 1]], ssem[j],
                         add=True)

    def wait_scatter(j):
        pltpu.make_async_copy(rows[j], acc_sh.at[pl.ds(0, B)], ssem[j]).wait()

    plsc.subcore_barrier()
    start_gather(0, 0)

    iota = lax.iota(I32, 16)
    base8 = lax.shift_right_logical(iota, 3)

    def compute(j):
        rv = rows[j]
        av = alds[j]
        if layer == 1:
            @plsc.parallel_loop(0, B, 1, unroll=4)
            def edge(e):
                ald16 = av[e, :]
                srcal = rv[e, pl.ds(64, 16)]
                am = srcal + ald16
                am = jnp.maximum(am, am * NEG)
                ex = jnp.exp(am)
                rv[e, pl.ds(64, 16)] = ex
                for jj in range(4):
                    mult = jnp.take(ex, base8 + 2 * jj)
                    hj = rv[e, pl.ds(16 * jj, 16)]
                    rv[e, pl.ds(16 * jj, 16)] = hj * mult
        else:
            c12 = jnp.full((16,), 12, I32)

            @plsc.parallel_loop(0, B, 1, unroll=4)
            def edge(e):
                ald16 = av[e, :]
                t16 = rv[e, pl.ds(32, 16)]
                am = t16 + ald16
                am = jnp.maximum(am, am * NEG)
                ex = jnp.exp(am)
                mult = jnp.take(ex, c12)
                h0 = rv[e, pl.ds(0, 16)]
                rv[e, pl.ds(0, 16)] = h0 * mult
                h1 = rv[e, pl.ds(16, 16)]
                rv[e, pl.ds(16, 16)] = h1 * mult
                rv[e, pl.ds(32, 16)] = jnp.where(iota < 8, t16 * mult, mult)

    kw = jnp.where(c == 0, KA, KB)

    def body_t(t, carry):
        for j in range(3):
            k = 3 * t + j
            nb = (j + 1) % 3

            @pl.when(k + 1 < kw)
            def _prefetch():
                @pl.when(k >= 2)
                def _drain():
                    wait_scatter(nb)

                start_gather(k + 1, nb)

            wait_gather(j)
            compute(j)
            start_scatter(k, j)
        return carry

    lax.fori_loop(0, kw // 3, body_t, 0)
    for j in range(3):
        wait_scatter(j)
    plsc.subcore_barrier()
    for t in range(RPT // B):
        pltpu.sync_copy(acc_sh.at[pl.ds(rowbase + t * B, B)], r0)
        pltpu.sync_copy(r0, out.at[c, pl.ds(rowbase + t * B, B)])


def _sc_edge(layer):
    rw = RW1 if layer == 1 else RW2
    mesh = plsc.VectorSubcoreMesh(core_axis_name="c", subcore_axis_name="s")
    return pl.kernel(
        functools.partial(_sc_body, layer),
        out_type=jax.ShapeDtypeStruct((NC, NP, rw), F32),
        mesh=mesh,
        scratch_types=[
            pltpu.VMEM((KMAX, 2, B), I32),
            pltpu.VMEM((B, rw), F32),
            pltpu.VMEM((B, rw), F32),
            pltpu.VMEM((B, rw), F32),
            pltpu.VMEM((B, 16), F32),
            pltpu.VMEM((B, 16), F32),
            pltpu.VMEM((B, 16), F32),
            pltpu.VMEM_SHARED((NP, rw), F32),
            pltpu.SemaphoreType.DMA,
            pltpu.SemaphoreType.DMA,
            pltpu.SemaphoreType.DMA,
            pltpu.SemaphoreType.DMA,
            pltpu.SemaphoreType.DMA,
            pltpu.SemaphoreType.DMA,
        ],
        compiler_params=pltpu.CompilerParams(
            needs_layout_passes=False, use_tc_tiling_on_sc=False),
    )


def kernel(x, edge_index, W1, att_src1, att_dst1, b1, W2, att_src2, att_dst2,
           b2):
    loop = jnp.arange(N, dtype=I32)
    padv = jnp.full((EPAD - ETOT,), N, I32)
    sidx = jnp.concatenate([edge_index[0].astype(I32), loop, padv])
    didx = jnp.concatenate([edge_index[1].astype(I32), loop, padv])

    def _split(v):
        ec0 = NS * KA * B
        v0 = v[:ec0].reshape(NS, KA, B)
        v0 = jnp.pad(v0, ((0, 0), (0, KMAX - KA), (0, 0)),
                     constant_values=N)
        v1 = v[ec0:].reshape(NS, KB, B)
        v1 = jnp.pad(v1, ((0, 0), (0, KMAX - KB), (0, 0)),
                     constant_values=N)
        return jnp.concatenate([v0, v1], axis=0)

    sdidx = jnp.stack([_split(sidx), _split(didx)], axis=2)
    xp = jnp.pad(x, ((0, NP - N), (0, 0)))

    eye8 = jnp.eye(8, dtype=F32)
    as64 = (att_src1[:, :, None] * eye8[:, None, :]).reshape(64, 8)
    ad64 = (att_dst1[:, :, None] * eye8[:, None, :]).reshape(64, 8)

    table1, aldt1 = pl.pallas_call(
        _tc1_body,
        grid=(NP // BLK,),
        in_specs=[
            pl.BlockSpec((BLK, 128), lambda i: (i, 0)),
            pl.BlockSpec((128, 64), lambda i: (0, 0)),
            pl.BlockSpec((64, 8), lambda i: (0, 0)),
            pl.BlockSpec((64, 8), lambda i: (0, 0)),
        ],
        out_specs=[
            pl.BlockSpec((BLK, RW1), lambda i: (i, 0)),
            pl.BlockSpec((BLK, 16), lambda i: (i, 0)),
        ],
        out_shape=[
            jax.ShapeDtypeStruct((NP, RW1), F32),
            jax.ShapeDtypeStruct((NP, 16), F32),
        ],
    )(xp, W1, as64, ad64)

    part1 = _sc_edge(1)(sdidx, table1, aldt1)

    r8 = jnp.repeat(jnp.eye(8, dtype=F32), 8, axis=1)
    as2t = jnp.tile(att_src2.T, (1, 8))
    ad2t = jnp.tile(att_dst2.T, (1, 16))

    table2, aldt2 = pl.pallas_call(
        _tc2_body,
        grid=(NP // BLK,),
        in_specs=[
            pl.BlockSpec((NC, BLK, RW1), lambda i: (0, i, 0)),
            pl.BlockSpec((1, 64), lambda i: (0, 0)),
            pl.BlockSpec((64, 40), lambda i: (0, 0)),
            pl.BlockSpec((40, 8), lambda i: (0, 0)),
            pl.BlockSpec((40, 16), lambda i: (0, 0)),
            pl.BlockSpec((8, 64), lambda i: (0, 0)),
        ],
        out_specs=[
            pl.BlockSpec((BLK, RW2), lambda i: (i, 0)),
            pl.BlockSpec((BLK, 16), lambda i: (i, 0)),
        ],
        out_shape=[
            jax.ShapeDtypeStruct((NP, RW2), F32),
            jax.ShapeDtypeStruct((NP, 16), F32),
        ],
    )(part1, b1.reshape(1, 64), W2, as2t, ad2t, r8)

    part2 = _sc_edge(2)(sdidx, table2, aldt2)

    rb = jnp.concatenate([jnp.ones((1, 40), F32), jnp.zeros((7, 40), F32)])
    out = pl.pallas_call(
        _tc3_body,
        grid=(25,),
        in_specs=[
            pl.BlockSpec((NC, 400, RW2), lambda i: (0, i, 0)),
            pl.BlockSpec((1, 40), lambda i: (0, 0)),
            pl.BlockSpec((8, 40), lambda i: (0, 0)),
        ],
        out_specs=pl.BlockSpec((400, 40), lambda i: (i, 0)),
        out_shape=jax.ShapeDtypeStruct((N, 40), F32),
    )(part2, b2.reshape(1, 40), rb)
    return out


# speedup vs baseline: 1.6602x; 1.0475x over previous
import functools

import jax
import jax.numpy as jnp
from jax import lax
from jax.experimental import pallas as pl
from jax.experimental.pallas import tpu as pltpu
from jax.experimental.pallas import tpu_sc as plsc

F32 = jnp.float32
I32 = jnp.int32

N = 10000
NP = 10240
E_IN = 320000
ETOT = E_IN + N
NC, NS = 2, 16
NW = NC * NS
B = 128
K = -(-ETOT // (NW * B))
KA = 96
KB = 2 * K - KA
KMAX = max(KA, KB)
EPAD = NW * B * K
RW1 = 80
RW2 = 48
NEG = 0.2
BLK = 512
RPT = NP // NS


def _tc1_body(x_ref, w1_ref, as_ref, ad_ref, t_ref, aldt_ref):
    h = jnp.dot(x_ref[...], w1_ref[...], preferred_element_type=F32)
    als = jnp.dot(h, as_ref[...], preferred_element_type=F32)
    ald = jnp.dot(h, ad_ref[...], preferred_element_type=F32)
    t_ref[...] = jnp.concatenate([h, als, als], axis=1)
    aldt_ref[...] = jnp.concatenate([ald, ald], axis=1)


def _tc2_body(p_ref, b1_ref, w2_ref, as2_ref, ad2_ref, r8_ref, t2_ref, ald2_ref):
    a = p_ref[0] + p_ref[1]
    den64 = jnp.dot(a[:, 64:72], r8_ref[...], preferred_element_type=F32)
    x2 = a[:, 0:64] / (den64 + 1e-16) + b1_ref[...]
    x2 = jnp.where(x2 > 0, x2, jnp.exp(x2) - 1.0)
    h2 = jnp.dot(x2, w2_ref[...], preferred_element_type=F32)
    als2 = jnp.dot(h2, as2_ref[...], preferred_element_type=F32)
    ald2 = jnp.dot(h2, ad2_ref[...], preferred_element_type=F32)
    t2_ref[...] = jnp.concatenate([h2, als2], axis=1)
    ald2_ref[...] = ald2


def _tc3_body(p_ref, b2_ref, rb_ref, o_ref):
    a = p_ref[0] + p_ref[1]
    den40 = jnp.dot(a[:, 40:48], rb_ref[...], preferred_element_type=F32)
    o_ref[...] = a[:, 0:40] / (den40 + 1e-16) + b2_ref[...] + 1e-6


def _sc_body(layer, sdidx, table, aldt, out, idx_all, r0, r1, r2, a0, a1, a2,
             acc_sh, g0, g1, g2, s0, s1, s2):
    rw = RW1 if layer == 1 else RW2
    rows = (r0, r1, r2)
    alds = (a0, a1, a2)
    gsem = (g0, g1, g2)
    ssem = (s0, s1, s2)
    c = lax.axis_index("c")
    s = lax.axis_index("s")
    w = c * NS + s
    rowbase = s * RPT

    pltpu.sync_copy(sdidx.at[w], idx_all)

    zero16 = jnp.zeros((16,), F32)

    def zrow(e, carry):
        for j in range(rw // 16):
            r0[e, pl.ds(16 * j, 16)] = zero16
        return carry

    lax.fori_loop(0, B, zrow, 0)
    for t in range(RPT // B):
        pltpu.sync_copy(r0, acc_sh.at[pl.ds(rowbase + t * B, B)])

    def start_gather(k, j):
        pltpu.async_copy(table.at[idx_all.at[k, 0]], rows[j], gsem[j])
        pltpu.async_copy(aldt.at[idx_all.at[k, 1]], alds[j], gsem[j])

    def wait_gather(j):
        pltpu.make_async_copy(table.at[pl.ds(0, B)], rows[j],
                              gsem[j]).wait()
        pltpu.make_async_copy(aldt.at[pl.ds(0, B)], alds[j], gsem[j]).wait()

    def start_scatter(k, j):
        pltpu.async_copy(rows[j], acc_sh.at[idx_all.at[k, 1]], ssem[j],
                         add=True)

    def wait_scatter(j):
        pltpu.make_async_copy(rows[j], acc_sh.at[pl.ds(0, B)], ssem[j]).wait()

    plsc.subcore_barrier()
    start_gather(0, 0)

    iota = lax.iota(I32, 16)
    base8 = lax.shift_right_logical(iota, 3)

    def compute(j):
        rv = rows[j]
        av = alds[j]
        if layer == 1:
            @plsc.parallel_loop(0, B, 1, unroll=4)
            def edge(e):
                ald16 = av[e, :]
                srcal = rv[e, pl.ds(64, 16)]
                am = srcal + ald16
                am = jnp.maximum(am, am * NEG)
                ex = jnp.exp(am)
                rv[e, pl.ds(64, 16)] = ex
                for jj in range(4):
                    mult = jnp.take(ex, base8 + 2 * jj)
                    hj = rv[e, pl.ds(16 * jj, 16)]
                    rv[e, pl.ds(16 * jj, 16)] = hj * mult
        else:
            c12 = jnp.full((16,), 12, I32)

            @plsc.parallel_loop(0, B, 1, unroll=4)
            def edge(e):
                ald16 = av[e, :]
                t16 = rv[e, pl.ds(32, 16)]
                am = t16 + ald16
                am = jnp.maximum(am, am * NEG)
                ex = jnp.exp(am)
                mult = jnp.take(ex, c12)
                h0 = rv[e, pl.ds(0, 16)]
                rv[e, pl.ds(0, 16)] = h0 * mult
                h1 = rv[e, pl.ds(16, 16)]
                rv[e, pl.ds(16, 16)] = h1 * mult
                rv[e, pl.ds(32, 16)] = jnp.where(iota < 8, t16 * mult, mult)

    kw = jnp.where(c == 0, KA, KB)

    def body_t(t, carry):
        for j in range(3):
            k = 3 * t + j
            nb = (j + 1) % 3

            @pl.when(k + 1 < kw)
            def _prefetch():
                @pl.when(k >= 2)
                def _drain():
                    wait_scatter(nb)

                start_gather(k + 1, nb)

            wait_gather(j)
            compute(j)
            start_scatter(k, j)
        return carry

    lax.fori_loop(0, kw // 3, body_t, 0)
    for j in range(3):
        wait_scatter(j)
    plsc.subcore_barrier()
    for t in range(RPT // B):
        pltpu.sync_copy(acc_sh.at[pl.ds(rowbase + t * B, B)], r0)
        pltpu.sync_copy(r0, out.at[c, pl.ds(rowbase + t * B, B)])


def _sc_edge(layer):
    rw = RW1 if layer == 1 else RW2
    mesh = plsc.VectorSubcoreMesh(core_axis_name="c", subcore_axis_name="s")
    return pl.kernel(
        functools.partial(_sc_body, layer),
        out_type=jax.ShapeDtypeStruct((NC, NP, rw), F32),
        mesh=mesh,
        scratch_types=[
            pltpu.VMEM((KMAX, 2, B), I32),
            pltpu.VMEM((B, rw), F32),
            pltpu.VMEM((B, rw), F32),
            pltpu.VMEM((B, rw), F32),
            pltpu.VMEM((B, 16), F32),
            pltpu.VMEM((B, 16), F32),
            pltpu.VMEM((B, 16), F32),
            pltpu.VMEM_SHARED((NP, rw), F32),
            pltpu.SemaphoreType.DMA,
            pltpu.SemaphoreType.DMA,
            pltpu.SemaphoreType.DMA,
            pltpu.SemaphoreType.DMA,
            pltpu.SemaphoreType.DMA,
            pltpu.SemaphoreType.DMA,
        ],
        compiler_params=pltpu.CompilerParams(
            needs_layout_passes=False, use_tc_tiling_on_sc=False),
    )


def kernel(x, edge_index, W1, att_src1, att_dst1, b1, W2, att_src2, att_dst2,
           b2):
    loop = jnp.arange(N, dtype=I32)
    padv = jnp.full((EPAD - ETOT,), N, I32)
    sidx = jnp.concatenate([edge_index[0].astype(I32), loop, padv])
    didx = jnp.concatenate([edge_index[1].astype(I32), loop, padv])

    def _split(v):
        ec0 = NS * KA * B
        v0 = v[:ec0].reshape(NS, KA, B)
        v0 = jnp.pad(v0, ((0, 0), (0, KMAX - KA), (0, 0)),
                     constant_values=N)
        v1 = v[ec0:].reshape(NS, KB, B)
        v1 = jnp.pad(v1, ((0, 0), (0, KMAX - KB), (0, 0)),
                     constant_values=N)
        return jnp.concatenate([v0, v1], axis=0)

    sdidx = jnp.stack([_split(sidx), _split(didx)], axis=2)
    xp = jnp.pad(x, ((0, NP - N), (0, 0)))

    eye8 = jnp.eye(8, dtype=F32)
    as64 = (att_src1[:, :, None] * eye8[:, None, :]).reshape(64, 8)
    ad64 = (att_dst1[:, :, None] * eye8[:, None, :]).reshape(64, 8)

    table1, aldt1 = pl.pallas_call(
        _tc1_body,
        grid=(NP // BLK,),
        in_specs=[
            pl.BlockSpec((BLK, 128), lambda i: (i, 0)),
            pl.BlockSpec((128, 64), lambda i: (0, 0)),
            pl.BlockSpec((64, 8), lambda i: (0, 0)),
            pl.BlockSpec((64, 8), lambda i: (0, 0)),
        ],
        out_specs=[
            pl.BlockSpec((BLK, RW1), lambda i: (i, 0)),
            pl.BlockSpec((BLK, 16), lambda i: (i, 0)),
        ],
        out_shape=[
            jax.ShapeDtypeStruct((NP, RW1), F32),
            jax.ShapeDtypeStruct((NP, 16), F32),
        ],
    )(xp, W1, as64, ad64)

    part1 = _sc_edge(1)(sdidx, table1, aldt1)

    r8 = jnp.repeat(jnp.eye(8, dtype=F32), 8, axis=1)
    as2t = jnp.tile(att_src2.T, (1, 8))
    ad2t = jnp.tile(att_dst2.T, (1, 16))

    table2, aldt2 = pl.pallas_call(
        _tc2_body,
        grid=(NP // BLK,),
        in_specs=[
            pl.BlockSpec((NC, BLK, RW1), lambda i: (0, i, 0)),
            pl.BlockSpec((1, 64), lambda i: (0, 0)),
            pl.BlockSpec((64, 40), lambda i: (0, 0)),
            pl.BlockSpec((40, 8), lambda i: (0, 0)),
            pl.BlockSpec((40, 16), lambda i: (0, 0)),
            pl.BlockSpec((8, 64), lambda i: (0, 0)),
        ],
        out_specs=[
            pl.BlockSpec((BLK, RW2), lambda i: (i, 0)),
            pl.BlockSpec((BLK, 16), lambda i: (i, 0)),
        ],
        out_shape=[
            jax.ShapeDtypeStruct((NP, RW2), F32),
            jax.ShapeDtypeStruct((NP, 16), F32),
        ],
    )(part1, b1.reshape(1, 64), W2, as2t, ad2t, r8)

    part2 = _sc_edge(2)(sdidx, table2, aldt2)

    rb = jnp.concatenate([jnp.ones((1, 40), F32), jnp.zeros((7, 40), F32)])
    out = pl.pallas_call(
        _tc3_body,
        grid=(25,),
        in_specs=[
            pl.BlockSpec((NC, 400, RW2), lambda i: (0, i, 0)),
            pl.BlockSpec((1, 40), lambda i: (0, 0)),
            pl.BlockSpec((8, 40), lambda i: (0, 0)),
        ],
        out_specs=pl.BlockSpec((400, 40), lambda i: (i, 0)),
        out_shape=jax.ShapeDtypeStruct((N, 40), F32),
    )(part2, b2.reshape(1, 40), rb)
    return out
